# baseline (speedup 1.0000x reference)
import functools
import jax
import jax.numpy as jnp
from jax import lax
from jax.experimental import pallas as pl
from jax.experimental.pallas import tpu as pltpu
from jax.experimental.pallas import tpu_sc as plsc

N = 10000
E = 320000
D = 128

NC = 2
NS = 16
L = 16
NPAD = 10240
CH = 128
NCHT = E // CH
KMAIN = NCHT // NS
NEPI = NCHT - KMAIN * NS
RPT = NPAD // NS
ZR = 128


def _tca_body(x_ref, wz_ref, lz_ref, wh_ref, lh_ref, out_ref):
    mz = jnp.dot(wz_ref[...], lz_ref[...], preferred_element_type=jnp.float32)
    mh = jnp.dot(wh_ref[...], lh_ref[...], preferred_element_type=jnp.float32)
    xb = x_ref[...]
    out_ref[0] = jnp.dot(xb, mz, preferred_element_type=jnp.float32)
    out_ref[1] = jnp.dot(xb, mh, preferred_element_type=jnp.float32)


def _tc_a(x, Wz, Lz_W, Wh, Lh_W):
    BN = 2000
    grid = (N // BN,)
    return pl.pallas_call(
        _tca_body,
        grid=grid,
        in_specs=[
            pl.BlockSpec((BN, D), lambda i: (i, 0)),
            pl.BlockSpec((D, D), lambda i: (0, 0)),
            pl.BlockSpec((D, D), lambda i: (0, 0)),
            pl.BlockSpec((D, D), lambda i: (0, 0)),
            pl.BlockSpec((D, D), lambda i: (0, 0)),
        ],
        out_specs=pl.BlockSpec((2, BN, D), lambda i: (0, i, 0)),
        out_shape=jax.ShapeDtypeStruct((2, N, D), jnp.float32),
    )(x, Wz, Lz_W, Wh, Lh_W)


def _sc_body(src_hbm, dst_hbm, w_hbm, u_hbm,
             s_hbm, dis_hbm,
             dis_sp, agg_sp,
             srcb0, srcb1, dstb0, dstb1, wb0, wb1, adjb0, adjb1, fb0, fb1,
             sdst0, sdst1, dstbig, rows0, rows1,
             dis_tile, slbuf,
             semi0, semi1, semg0, semg1, sems0, sems1):
    c = lax.axis_index("c")
    s = lax.axis_index("s")
    zero16 = jnp.zeros((L,), jnp.float32)
    srcb = (srcb0, srcb1)
    dstb = (dstb0, dstb1)
    wb = (wb0, wb1)
    adjb = (adjb0, adjb1)
    fb = (fb0, fb1)
    sdst = (sdst0, sdst1)
    rows = (rows0, rows1)
    semi = (semi0, semi1)
    semg = (semg0, semg1)
    sems = (sems0, sems1)

    def ebase(k):
        return (s + NS * k) * CH

    def _zero_rows0(i, carry):
        for j in range(D // L):
            rows0[i, pl.ds(j * L, L)] = zero16
        return carry
    lax.fori_loop(0, ZR, _zero_rows0, 0)

    EB = 512
    EPT = E // NS
    NDG = EPT // EB
    EBT = EPT - NDG * EB

    def degb_issue(k, b, ne=EB):
        e0 = s * EPT + k * EB
        pltpu.async_copy(dst_hbm.at[pl.ds(e0, ne)],
                         dstbig.at[pl.ds(EB * b, ne)], semi[b])
        pltpu.async_copy(w_hbm.at[pl.ds(e0, ne)],
                         dis_tile.at[pl.ds(EB * b, ne)], semi[b])

    def degb_wait(b, ne=EB):
        pltpu.make_async_copy(dst_hbm.at[pl.ds(0, ne)],
                              dstbig.at[pl.ds(EB * b, ne)], semi[b]).wait()
        pltpu.make_async_copy(w_hbm.at[pl.ds(0, ne)],
                              dis_tile.at[pl.ds(EB * b, ne)], semi[b]).wait()

    def degb_accum(b, ne=EB):
        def _vec(t, carry):
            dv = dstbig[pl.ds(EB * b + t * L, L)]
            wv = dis_tile[pl.ds(EB * b + t * L, L)]
            rv = lax.shift_right_logical(dv, 7)
            cv = lax.bitwise_and(dv, 127)
            plsc.addupdate_scatter(rows0, [rv, cv], wv)
            return carry
        lax.fori_loop(0, ne // L, _vec, 0)

    degb_issue(0, 0)
    degb_issue(1, 1)
    for k in range(NDG):
        b = k % 2
        degb_wait(b)
        degb_accum(b)
        if k + 2 < NDG:
            degb_issue(k + 2, b)
        elif k + 2 == NDG:
            degb_issue(NDG, b, ne=EBT)
    bt = NDG % 2
    degb_wait(bt, ne=EBT)
    degb_accum(bt, ne=EBT)

    @pl.when(s < NEPI)
    def _deg_epilogue():
        e0 = NS * EPT + s * CH
        pltpu.sync_copy(dst_hbm.at[pl.ds(e0, CH)], dstbig.at[pl.ds(0, CH)])
        pltpu.sync_copy(w_hbm.at[pl.ds(e0, CH)], dis_tile.at[pl.ds(0, CH)])
        degb_accum(0, ne=CH)

    pltpu.sync_copy(rows0.at[pl.ds(0, 80)], agg_sp.at[pl.ds(s * 80, 80)])
    plsc.subcore_barrier()

    for h in range(NS):
        pltpu.sync_copy(agg_sp.at[pl.ds(h * 80 + 5 * s, 5)],
                        rows1.at[pl.ds(5 * h, 5)])
    def _reduce_row(r, carry):
        for j in range(D // L):
            acc = rows1[r, pl.ds(j * L, L)]
            for h in range(1, NS):
                acc = acc + rows1[5 * h + r, pl.ds(j * L, L)]
            dv = acc + 1.0
            bits = lax.bitcast_convert_type(dv, jnp.int32)
            y = lax.bitcast_convert_type(jnp.int32(0x5F3759DF) - (bits >> 1),
                                         jnp.float32)
            xh = dv * 0.5
            y = y * (1.5 - xh * y * y)
            y = y * (1.5 - xh * y * y)
            y = y * (1.5 - xh * y * y)
            y = jnp.where(dv > 0.0, y, 0.0)
            slbuf[pl.ds(r * D + j * L, L)] = y
        return carry
    lax.fori_loop(0, 5, _reduce_row, 0)

    pltpu.sync_copy(slbuf, dis_sp.at[pl.ds(s * 640, 640)])
    pltpu.sync_copy(slbuf, dis_hbm.at[c, pl.ds(s * 640, 640)])
    plsc.subcore_barrier()

    lax.fori_loop(0, ZR, _zero_rows0, 0)
    for k in range(RPT // ZR):
        pltpu.sync_copy(rows0, agg_sp.at[pl.ds(s * RPT + k * ZR, ZR)])
    pltpu.sync_copy(dis_sp, dis_tile)
    plsc.subcore_barrier()

    coff = jnp.zeros((L,), jnp.int32) + c * N

    def idx_issue(k, b):
        e0 = ebase(k)
        pltpu.async_copy(src_hbm.at[pl.ds(e0, CH)], srcb[b], semi[b])
        pltpu.async_copy(dst_hbm.at[pl.ds(e0, CH)], dstb[b], semi[b])
        pltpu.async_copy(w_hbm.at[pl.ds(e0, CH)], wb[b], semi[b])

    def idx_wait(b):
        pltpu.make_async_copy(src_hbm.at[pl.ds(0, CH)], srcb[b], semi[b]).wait()
        pltpu.make_async_copy(dst_hbm.at[pl.ds(0, CH)], dstb[b], semi[b]).wait()
        pltpu.make_async_copy(w_hbm.at[pl.ds(0, CH)], wb[b], semi[b]).wait()

    def compute_factors(b):
        for j in range(CH // L):
            sv = srcb[b][pl.ds(j * L, L)]
            adjb[b][pl.ds(j * L, L)] = sv + coff
            dsv = plsc.load_gather(dis_tile, [sv])
            fb[b][pl.ds(j * L, L)] = dsv * wb[b][pl.ds(j * L, L)]

    def scatter_wait(b):
        pltpu.make_async_copy(rows[b].at[pl.ds(0, CH // 2)],
                              agg_sp.at[pl.ds(0, CH // 2)], sems[b]).wait()
        pltpu.make_async_copy(rows[b].at[pl.ds(0, CH // 2)],
                              agg_sp.at[pl.ds(0, CH // 2)], sems[b]).wait()

    def gather_start(b, wait=True):
        if wait:
            scatter_wait(b)
        pltpu.async_copy(u_hbm.at[adjb[b]], rows[b], semg[b])

    def gather_wait(b):
        pltpu.make_async_copy(u_hbm.at[pl.ds(0, CH)], rows[b], semg[b]).wait()

    UNROLL = 4

    HH = CH // 2

    def scale_scatter(b):
        for j in range(CH // L):
            sdst[b][j * L // HH, pl.ds((j * L) % HH, L)] = (
                dstb[b][pl.ds(j * L, L)])

        def _scale_rows(t, icarry):
            fvec = fb[b][pl.ds(t * L, L)]
            i0 = t * L
            for r in range(L):
                fv = jnp.take_along_axis(
                    fvec, jnp.full((L,), r, jnp.int32), axis=0)
                for j in range(D // L):
                    rows[b][i0 + r, pl.ds(j * L, L)] = (
                        rows[b][i0 + r, pl.ds(j * L, L)] * fv)
            return icarry
        lax.fori_loop(0, HH // L, _scale_rows, 0)
        pltpu.async_copy(rows[b].at[pl.ds(0, HH)], agg_sp.at[sdst[b].at[0]],
                         sems[b], add=True)
        lax.fori_loop(HH // L, CH // L, _scale_rows, 0)
        pltpu.async_copy(rows[b].at[pl.ds(HH, HH)], agg_sp.at[sdst[b].at[1]],
                         sems[b], add=True)

    idx_issue(0, 0)
    idx_issue(1, 1)
    idx_wait(0)
    compute_factors(0)
    gather_start(0, wait=False)

    def _pair_body(g, first):
        idx_wait(1)
        compute_factors(1)
        gather_start(1, wait=not first)
        gather_wait(0)
        scale_scatter(0)
        idx_issue(2 * g + 2, 0)
        idx_wait(0)
        compute_factors(0)
        gather_start(0)
        gather_wait(1)
        scale_scatter(1)
        idx_issue(2 * g + 3, 1)

    _pair_body(0, True)

    def _agg_pair(g, carry):
        _pair_body(g, False)
        return carry
    lax.fori_loop(1, KMAIN // 2 - 1, _agg_pair, 0)
    idx_wait(1)
    compute_factors(1)
    gather_start(1)
    gather_wait(0)
    scale_scatter(0)
    gather_wait(1)
    scale_scatter(1)
    scatter_wait(0)
    scatter_wait(1)

    @pl.when(s < NEPI)
    def _agg_epilogue():
        e0 = (KMAIN * NS + s) * CH
        pltpu.sync_copy(src_hbm.at[pl.ds(e0, CH)], srcb0)
        pltpu.sync_copy(dst_hbm.at[pl.ds(e0, CH)], dstb0)
        pltpu.sync_copy(w_hbm.at[pl.ds(e0, CH)], wb0)
        compute_factors(0)
        pltpu.async_copy(u_hbm.at[adjb0], rows0, semg0).wait()
        scale_scatter(0)
        scatter_wait(0)
    plsc.subcore_barrier()

    pltpu.sync_copy(agg_sp.at[pl.ds(s * RPT, RPT)],
                    s_hbm.at[c, pl.ds(s * RPT, RPT)])


def _sc_agg(src, dst, w, u_flat):
    mesh = plsc.VectorSubcoreMesh(core_axis_name="c", subcore_axis_name="s")
    kern = functools.partial(
        pl.kernel,
        out_type=[
            jax.ShapeDtypeStruct((NC, NPAD, D), jnp.float32),
            jax.ShapeDtypeStruct((NC, NPAD), jnp.float32),
        ],
        mesh=mesh,
        compiler_params=pltpu.CompilerParams(needs_layout_passes=False),
        scratch_types=[
            pltpu.VMEM_SHARED((NPAD,), jnp.float32),
            pltpu.VMEM_SHARED((NPAD, D), jnp.float32),
            pltpu.VMEM((CH,), jnp.int32),
            pltpu.VMEM((CH,), jnp.int32),
            pltpu.VMEM((CH,), jnp.int32),
            pltpu.VMEM((CH,), jnp.int32),
            pltpu.VMEM((CH,), jnp.float32),
            pltpu.VMEM((CH,), jnp.float32),
            pltpu.VMEM((CH,), jnp.int32),
            pltpu.VMEM((CH,), jnp.int32),
            pltpu.VMEM((CH,), jnp.float32),
            pltpu.VMEM((CH,), jnp.float32),
            pltpu.VMEM((2, CH // 2), jnp.int32),
            pltpu.VMEM((2, CH // 2), jnp.int32),
            pltpu.VMEM((2 * 512,), jnp.int32),
            pltpu.VMEM((CH, D), jnp.float32),
            pltpu.VMEM((CH, D), jnp.float32),
            pltpu.VMEM((NPAD,), jnp.float32),
            pltpu.VMEM((640,), jnp.float32),
            pltpu.SemaphoreType.DMA,
            pltpu.SemaphoreType.DMA,
            pltpu.SemaphoreType.DMA,
            pltpu.SemaphoreType.DMA,
            pltpu.SemaphoreType.DMA,
            pltpu.SemaphoreType.DMA,
        ],
    )(_sc_body)
    return kern(src, dst, w, u_flat)


def _tcc_body(sz_ref, sh_ref, dis_ref, uz_ref, uh_ref, hs_ref,
              lzw_ref, bz_ref, lzb_ref, lhw_ref, bh_ref, lhb_ref,
              wft_ref, wfb_ref, bf_ref, wc1_ref, bc1_ref, wc2_ref, bc2_ref,
              out_ref):
    dv = dis_ref[...]
    dd = dv * dv
    uz = uz_ref[...]
    uh = uh_ref[...]
    cz = jnp.dot(bz_ref[...].reshape(1, D), lzw_ref[...],
                 preferred_element_type=jnp.float32)[0] + lzb_ref[...]
    chh = jnp.dot(bh_ref[...].reshape(1, D), lhw_ref[...],
                  preferred_element_type=jnp.float32)[0] + lhb_ref[...]
    aggz = dv * sz_ref[0] + dd * uz
    aggh = dv * sh_ref[0] + dd * uh
    zg = jax.nn.sigmoid(aggz + cz)
    ht = jnp.tanh(aggh + chh)
    hd = (1.0 - zg) * ht
    hf = jnp.dot(hs_ref[...], wft_ref[...], preferred_element_type=jnp.float32)
    hf = hf + jnp.dot(hd, wfb_ref[...], preferred_element_type=jnp.float32)
    hf = jax.nn.relu(hf + bf_ref[...])
    hid = jax.nn.relu(jnp.dot(hf, wc1_ref[...],
                              preferred_element_type=jnp.float32) + bc1_ref[...])
    out_ref[...] = jax.nn.sigmoid(
        jnp.dot(hid, wc2_ref[...], preferred_element_type=jnp.float32)
        + bc2_ref[...])


def _tc_c(S, dis_col, u_flat, h_static, Lz_W, bz, Lz_b, Lh_W, bh, Lh_b,
          Wf, bf, Wc1, bc1, Wc2, bc2):
    BN = 1000
    grid = (N // BN,)

    def full(shape):
        return pl.BlockSpec(shape, lambda i: tuple(0 for _ in shape))

    return pl.pallas_call(
        _tcc_body,
        grid=grid,
        in_specs=[
            pl.BlockSpec((1, BN, D), lambda i: (0, i, 0)),
            pl.BlockSpec((1, BN, D), lambda i: (1, i, 0)),
            pl.BlockSpec((BN, 1), lambda i: (i, 0)),
            pl.BlockSpec((BN, D), lambda i: (i, 0)),
            pl.BlockSpec((BN, D), lambda i: (N // BN + i, 0)),
            pl.BlockSpec((BN, D), lambda i: (i, 0)),
            pl.BlockSpec((D, D), lambda i: (0, 0)),
            full((D,)), full((D,)),
            pl.BlockSpec((D, D), lambda i: (0, 0)),
            full((D,)), full((D,)),
            pl.BlockSpec((D, D), lambda i: (0, 0)),
            pl.BlockSpec((D, D), lambda i: (1, 0)),
            full((D,)),
            full((D, 64)), full((64,)), full((64, 1)), full((1,)),
        ],
        out_specs=pl.BlockSpec((BN, 1), lambda i: (i, 0)),
        out_shape=jax.ShapeDtypeStruct((N, 1), jnp.float32),
    )(S, S, dis_col, u_flat, u_flat, h_static,
      Lz_W, bz, Lz_b, Lh_W, bh, Lh_b, Wf, Wf, bf, Wc1, bc1, Wc2, bc2)


def kernel(x, edge_index, edge_attr, h_static,
           Wz, bz, Lz_W, Lz_b, Wr, br, Lr_W, Lr_b,
           Wh, bh, Lh_W, Lh_b, Wf, bf, Wc1, bc1, Wc2, bc2):
    src = edge_index[0]
    dst = edge_index[1]
    U = _tc_a(x, Wz, Lz_W, Wh, Lh_W)
    u_flat = U.reshape(2 * N, D)
    S, dis2 = _sc_agg(src, dst, edge_attr, u_flat)
    dis_col = dis2[0, :N].reshape(N, 1)
    return _tc_c(S, dis_col, u_flat, h_static,
                 Lz_W, bz, Lz_b, Lh_W, bh, Lh_b,
                 Wf, bf, Wc1, bc1, Wc2, bc2)


# speedup vs baseline: 1.0271x; 1.0271x over previous
import functools
import jax
import jax.numpy as jnp
from jax import lax
from jax.experimental import pallas as pl
from jax.experimental.pallas import tpu as pltpu
from jax.experimental.pallas import tpu_sc as plsc

N = 10000
E = 320000
D = 128

NC = 2
NS = 16
L = 16
NPAD = 10240
CH = 128
NCHT = E // CH
KMAIN = NCHT // NS
NEPI = NCHT - KMAIN * NS
RPT = NPAD // NS
ZR = 128


def _tca_body(x_ref, wz_ref, lz_ref, wh_ref, lh_ref, out_ref):
    mz = jnp.dot(wz_ref[...], lz_ref[...], preferred_element_type=jnp.float32)
    mh = jnp.dot(wh_ref[...], lh_ref[...], preferred_element_type=jnp.float32)
    xb = x_ref[...]
    out_ref[0] = jnp.dot(xb, mz, preferred_element_type=jnp.float32)
    out_ref[1] = jnp.dot(xb, mh, preferred_element_type=jnp.float32)


def _tc_a(x, Wz, Lz_W, Wh, Lh_W):
    BN = N
    grid = (N // BN,)
    return pl.pallas_call(
        _tca_body,
        grid=grid,
        in_specs=[
            pl.BlockSpec((BN, D), lambda i: (i, 0)),
            pl.BlockSpec((D, D), lambda i: (0, 0)),
            pl.BlockSpec((D, D), lambda i: (0, 0)),
            pl.BlockSpec((D, D), lambda i: (0, 0)),
            pl.BlockSpec((D, D), lambda i: (0, 0)),
        ],
        out_specs=pl.BlockSpec((2, BN, D), lambda i: (0, i, 0)),
        out_shape=jax.ShapeDtypeStruct((2, N, D), jnp.float32),
    )(x, Wz, Lz_W, Wh, Lh_W)


def _sc_body(src_hbm, dst_hbm, w_hbm, u_hbm,
             s_hbm, dis_hbm,
             dis_sp, agg_sp,
             srcb0, srcb1, dstb0, dstb1, wb0, wb1, adjb0, adjb1, fb0, fb1,
             sdst0, sdst1, dstbig, rows0, rows1,
             dis_tile, slbuf,
             semi0, semi1, semg0, semg1, sems0, sems1):
    c = lax.axis_index("c")
    s = lax.axis_index("s")
    zero16 = jnp.zeros((L,), jnp.float32)
    srcb = (srcb0, srcb1)
    dstb = (dstb0, dstb1)
    wb = (wb0, wb1)
    adjb = (adjb0, adjb1)
    fb = (fb0, fb1)
    sdst = (sdst0, sdst1)
    rows = (rows0, rows1)
    semi = (semi0, semi1)
    semg = (semg0, semg1)
    sems = (sems0, sems1)

    def ebase(k):
        return (s + NS * k) * CH

    def _zero_rows0(i, carry):
        for j in range(D // L):
            rows0[i, pl.ds(j * L, L)] = zero16
        return carry
    lax.fori_loop(0, ZR, _zero_rows0, 0)

    EB = 1024
    EPT = E // NS
    NDG = EPT // EB
    EBT = EPT - NDG * EB

    def degb_issue(k, b, ne=EB):
        e0 = s * EPT + k * EB
        pltpu.async_copy(dst_hbm.at[pl.ds(e0, ne)],
                         dstbig.at[pl.ds(EB * b, ne)], semi[b])
        pltpu.async_copy(w_hbm.at[pl.ds(e0, ne)],
                         dis_tile.at[pl.ds(EB * b, ne)], semi[b])

    def degb_wait(b, ne=EB):
        pltpu.make_async_copy(dst_hbm.at[pl.ds(0, ne)],
                              dstbig.at[pl.ds(EB * b, ne)], semi[b]).wait()
        pltpu.make_async_copy(w_hbm.at[pl.ds(0, ne)],
                              dis_tile.at[pl.ds(EB * b, ne)], semi[b]).wait()

    def degb_accum(b, ne=EB):
        def _vec(t, carry):
            dv = dstbig[pl.ds(EB * b + t * L, L)]
            wv = dis_tile[pl.ds(EB * b + t * L, L)]
            rv = lax.shift_right_logical(dv, 7)
            cv = lax.bitwise_and(dv, 127)
            plsc.addupdate_scatter(rows0, [rv, cv], wv)
            return carry
        lax.fori_loop(0, ne // L, _vec, 0)

    degb_issue(0, 0)
    degb_issue(1, 1)
    for k in range(NDG):
        b = k % 2
        degb_wait(b)
        degb_accum(b)
        if k + 2 < NDG:
            degb_issue(k + 2, b)
        elif k + 2 == NDG:
            degb_issue(NDG, b, ne=EBT)
    bt = NDG % 2
    degb_wait(bt, ne=EBT)
    degb_accum(bt, ne=EBT)

    @pl.when(s < NEPI)
    def _deg_epilogue():
        e0 = NS * EPT + s * CH
        pltpu.sync_copy(dst_hbm.at[pl.ds(e0, CH)], dstbig.at[pl.ds(0, CH)])
        pltpu.sync_copy(w_hbm.at[pl.ds(e0, CH)], dis_tile.at[pl.ds(0, CH)])
        degb_accum(0, ne=CH)

    pltpu.sync_copy(rows0.at[pl.ds(0, 80)], agg_sp.at[pl.ds(s * 80, 80)])
    plsc.subcore_barrier()

    for h in range(NS):
        pltpu.sync_copy(agg_sp.at[pl.ds(h * 80 + 5 * s, 5)],
                        rows1.at[pl.ds(5 * h, 5)])
    def _reduce_row(r, carry):
        for j in range(D // L):
            acc = rows1[r, pl.ds(j * L, L)]
            for h in range(1, NS):
                acc = acc + rows1[5 * h + r, pl.ds(j * L, L)]
            dv = acc + 1.0
            bits = lax.bitcast_convert_type(dv, jnp.int32)
            y = lax.bitcast_convert_type(jnp.int32(0x5F3759DF) - (bits >> 1),
                                         jnp.float32)
            xh = dv * 0.5
            y = y * (1.5 - xh * y * y)
            y = y * (1.5 - xh * y * y)
            y = y * (1.5 - xh * y * y)
            y = jnp.where(dv > 0.0, y, 0.0)
            slbuf[pl.ds(r * D + j * L, L)] = y
        return carry
    lax.fori_loop(0, 5, _reduce_row, 0)

    pltpu.sync_copy(slbuf, dis_sp.at[pl.ds(s * 640, 640)])
    pltpu.sync_copy(slbuf, dis_hbm.at[c, pl.ds(s * 640, 640)])
    plsc.subcore_barrier()

    lax.fori_loop(0, ZR, _zero_rows0, 0)
    for k in range(RPT // ZR):
        pltpu.sync_copy(rows0, agg_sp.at[pl.ds(s * RPT + k * ZR, ZR)])
    pltpu.sync_copy(dis_sp, dis_tile)
    plsc.subcore_barrier()

    coff = jnp.zeros((L,), jnp.int32) + c * N

    def idx_issue(k, b):
        e0 = ebase(k)
        pltpu.async_copy(src_hbm.at[pl.ds(e0, CH)], srcb[b], semi[b])
        pltpu.async_copy(dst_hbm.at[pl.ds(e0, CH)], dstb[b], semi[b])
        pltpu.async_copy(w_hbm.at[pl.ds(e0, CH)], wb[b], semi[b])

    def idx_wait(b):
        pltpu.make_async_copy(src_hbm.at[pl.ds(0, CH)], srcb[b], semi[b]).wait()
        pltpu.make_async_copy(dst_hbm.at[pl.ds(0, CH)], dstb[b], semi[b]).wait()
        pltpu.make_async_copy(w_hbm.at[pl.ds(0, CH)], wb[b], semi[b]).wait()

    def compute_factors(b):
        for j in range(CH // L):
            sv = srcb[b][pl.ds(j * L, L)]
            adjb[b][pl.ds(j * L, L)] = sv + coff
            dsv = plsc.load_gather(dis_tile, [sv])
            fb[b][pl.ds(j * L, L)] = dsv * wb[b][pl.ds(j * L, L)]

    def scatter_wait(b):
        pltpu.make_async_copy(rows[b], agg_sp.at[pl.ds(0, CH)],
                              sems[b]).wait()

    def gather_start(b, wait=True):
        if wait:
            scatter_wait(b)
        pltpu.async_copy(u_hbm.at[adjb[b]], rows[b], semg[b])

    def gather_wait(b):
        pltpu.make_async_copy(u_hbm.at[pl.ds(0, CH)], rows[b], semg[b]).wait()

    UNROLL = 4

    def scale_scatter(b):
        for j in range(CH // L):
            sdst[b][pl.ds(j * L, L)] = dstb[b][pl.ds(j * L, L)]

        def _scale_rows(t, icarry):
            fvec = fb[b][pl.ds(t * L, L)]
            i0 = t * L
            for r in range(L):
                fv = jnp.take_along_axis(
                    fvec, jnp.full((L,), r, jnp.int32), axis=0)
                for j in range(D // L):
                    rows[b][i0 + r, pl.ds(j * L, L)] = (
                        rows[b][i0 + r, pl.ds(j * L, L)] * fv)
            return icarry
        lax.fori_loop(0, CH // L, _scale_rows, 0)
        pltpu.async_copy(rows[b], agg_sp.at[sdst[b]], sems[b], add=True)

    idx_issue(0, 0)
    idx_issue(1, 1)
    idx_wait(0)
    compute_factors(0)
    gather_start(0, wait=False)

    def _pair_body(g, first):
        idx_wait(1)
        compute_factors(1)
        gather_start(1, wait=not first)
        gather_wait(0)
        scale_scatter(0)
        idx_issue(2 * g + 2, 0)
        idx_wait(0)
        compute_factors(0)
        gather_start(0)
        gather_wait(1)
        scale_scatter(1)
        idx_issue(2 * g + 3, 1)

    _pair_body(0, True)

    def _agg_pair(g, carry):
        _pair_body(g, False)
        return carry
    lax.fori_loop(1, KMAIN // 2 - 1, _agg_pair, 0)
    idx_wait(1)
    compute_factors(1)
    gather_start(1)
    gather_wait(0)
    scale_scatter(0)
    gather_wait(1)
    scale_scatter(1)
    scatter_wait(0)
    scatter_wait(1)

    @pl.when(s < NEPI)
    def _agg_epilogue():
        e0 = (KMAIN * NS + s) * CH
        pltpu.sync_copy(src_hbm.at[pl.ds(e0, CH)], srcb0)
        pltpu.sync_copy(dst_hbm.at[pl.ds(e0, CH)], dstb0)
        pltpu.sync_copy(w_hbm.at[pl.ds(e0, CH)], wb0)
        compute_factors(0)
        pltpu.async_copy(u_hbm.at[adjb0], rows0, semg0).wait()
        scale_scatter(0)
        scatter_wait(0)
    plsc.subcore_barrier()

    pltpu.sync_copy(agg_sp.at[pl.ds(s * RPT, RPT)],
                    s_hbm.at[c, pl.ds(s * RPT, RPT)])


def _sc_agg(src, dst, w, u_flat):
    mesh = plsc.VectorSubcoreMesh(core_axis_name="c", subcore_axis_name="s")
    kern = functools.partial(
        pl.kernel,
        out_type=[
            jax.ShapeDtypeStruct((NC, NPAD, D), jnp.float32),
            jax.ShapeDtypeStruct((NC, NPAD), jnp.float32),
        ],
        mesh=mesh,
        compiler_params=pltpu.CompilerParams(needs_layout_passes=False),
        scratch_types=[
            pltpu.VMEM_SHARED((NPAD,), jnp.float32),
            pltpu.VMEM_SHARED((NPAD, D), jnp.float32),
            pltpu.VMEM((CH,), jnp.int32),
            pltpu.VMEM((CH,), jnp.int32),
            pltpu.VMEM((CH,), jnp.int32),
            pltpu.VMEM((CH,), jnp.int32),
            pltpu.VMEM((CH,), jnp.float32),
            pltpu.VMEM((CH,), jnp.float32),
            pltpu.VMEM((CH,), jnp.int32),
            pltpu.VMEM((CH,), jnp.int32),
            pltpu.VMEM((CH,), jnp.float32),
            pltpu.VMEM((CH,), jnp.float32),
            pltpu.VMEM((CH,), jnp.int32),
            pltpu.VMEM((CH,), jnp.int32),
            pltpu.VMEM((2048,), jnp.int32),
            pltpu.VMEM((CH, D), jnp.float32),
            pltpu.VMEM((CH, D), jnp.float32),
            pltpu.VMEM((NPAD,), jnp.float32),
            pltpu.VMEM((640,), jnp.float32),
            pltpu.SemaphoreType.DMA,
            pltpu.SemaphoreType.DMA,
            pltpu.SemaphoreType.DMA,
            pltpu.SemaphoreType.DMA,
            pltpu.SemaphoreType.DMA,
            pltpu.SemaphoreType.DMA,
        ],
    )(_sc_body)
    return kern(src, dst, w, u_flat)


def _tcc_body(sz_ref, sh_ref, dis_ref, uz_ref, uh_ref, hs_ref,
              lzw_ref, bz_ref, lzb_ref, lhw_ref, bh_ref, lhb_ref,
              wft_ref, wfb_ref, bf_ref, wc1_ref, bc1_ref, wc2_ref, bc2_ref,
              out_ref):
    dv = dis_ref[...]
    dd = dv * dv
    uz = uz_ref[...]
    uh = uh_ref[...]
    cz = jnp.dot(bz_ref[...].reshape(1, D), lzw_ref[...],
                 preferred_element_type=jnp.float32)[0] + lzb_ref[...]
    chh = jnp.dot(bh_ref[...].reshape(1, D), lhw_ref[...],
                  preferred_element_type=jnp.float32)[0] + lhb_ref[...]
    aggz = dv * sz_ref[0] + dd * uz
    aggh = dv * sh_ref[0] + dd * uh
    zg = jax.nn.sigmoid(aggz + cz)
    ht = jnp.tanh(aggh + chh)
    hd = (1.0 - zg) * ht
    hf = jnp.dot(hs_ref[...], wft_ref[...], preferred_element_type=jnp.float32)
    hf = hf + jnp.dot(hd, wfb_ref[...], preferred_element_type=jnp.float32)
    hf = jax.nn.relu(hf + bf_ref[...])
    hid = jax.nn.relu(jnp.dot(hf, wc1_ref[...],
                              preferred_element_type=jnp.float32) + bc1_ref[...])
    out_ref[...] = jax.nn.sigmoid(
        jnp.dot(hid, wc2_ref[...], preferred_element_type=jnp.float32)
        + bc2_ref[...])


def _tc_c(S, dis_col, u_flat, h_static, Lz_W, bz, Lz_b, Lh_W, bh, Lh_b,
          Wf, bf, Wc1, bc1, Wc2, bc2):
    BN = N
    grid = (N // BN,)

    def full(shape):
        return pl.BlockSpec(shape, lambda i: tuple(0 for _ in shape))

    return pl.pallas_call(
        _tcc_body,
        grid=grid,
        in_specs=[
            pl.BlockSpec((1, BN, D), lambda i: (0, i, 0)),
            pl.BlockSpec((1, BN, D), lambda i: (1, i, 0)),
            pl.BlockSpec((BN, 1), lambda i: (i, 0)),
            pl.BlockSpec((BN, D), lambda i: (i, 0)),
            pl.BlockSpec((BN, D), lambda i: (N // BN + i, 0)),
            pl.BlockSpec((BN, D), lambda i: (i, 0)),
            pl.BlockSpec((D, D), lambda i: (0, 0)),
            full((D,)), full((D,)),
            pl.BlockSpec((D, D), lambda i: (0, 0)),
            full((D,)), full((D,)),
            pl.BlockSpec((D, D), lambda i: (0, 0)),
            pl.BlockSpec((D, D), lambda i: (1, 0)),
            full((D,)),
            full((D, 64)), full((64,)), full((64, 1)), full((1,)),
        ],
        out_specs=pl.BlockSpec((BN, 1), lambda i: (i, 0)),
        out_shape=jax.ShapeDtypeStruct((N, 1), jnp.float32),
    )(S, S, dis_col, u_flat, u_flat, h_static,
      Lz_W, bz, Lz_b, Lh_W, bh, Lh_b, Wf, Wf, bf, Wc1, bc1, Wc2, bc2)


def kernel(x, edge_index, edge_attr, h_static,
           Wz, bz, Lz_W, Lz_b, Wr, br, Lr_W, Lr_b,
           Wh, bh, Lh_W, Lh_b, Wf, bf, Wc1, bc1, Wc2, bc2):
    src = edge_index[0]
    dst = edge_index[1]
    U = _tc_a(x, Wz, Lz_W, Wh, Lh_W)
    u_flat = U.reshape(2 * N, D)
    S, dis2 = _sc_agg(src, dst, edge_attr, u_flat)
    dis_col = dis2[0, :N].reshape(N, 1)
    return _tc_c(S, dis_col, u_flat, h_static,
                 Lz_W, bz, Lz_b, Lh_W, bh, Lh_b,
                 Wf, bf, Wc1, bc1, Wc2, bc2)
